# baseline (speedup 1.0000x reference)
import functools

import jax
import jax.numpy as jnp
from jax import lax
from jax.experimental import pallas as pl
from jax.experimental.pallas import tpu as pltpu
from jax.experimental.pallas import tpu_sc as plsc

_M = 1_000_000
_D = 32
_B = 16384

_NC = 2
_NS = 16
_NW = _NC * _NS
_R = (_M // _NW) // 8 * 8
_REM = _M - _NW * _R
_RLAST = _R + _REM
_NV = _B // 16
_CR = 496
_NCP = _R // _CR
_NBUF = 4


def _sc_scatter_update(mem, val, idx):
  mesh = plsc.VectorSubcoreMesh(core_axis_name="c", subcore_axis_name="s")

  @functools.partial(
      pl.kernel,
      mesh=mesh,
      compiler_params=pltpu.CompilerParams(
          needs_layout_passes=False, use_tc_tiling_on_sc=False,
          disable_bounds_checks=True),
      out_type=jax.ShapeDtypeStruct((_M, _D), jnp.float32),
      scratch_types=[
          pltpu.VMEM((_B,), jnp.int32),
          pltpu.VMEM((_RLAST,), jnp.int32),
          pltpu.VMEM((_NBUF, _CR, _D), jnp.float32),
          pltpu.SemaphoreType.DMA,
          pltpu.SemaphoreType.DMA,
          pltpu.SemaphoreType.DMA,
          pltpu.SemaphoreType.DMA,
          pltpu.SemaphoreType.DMA,
          pltpu.SemaphoreType.DMA,
          pltpu.SemaphoreType.DMA,
          pltpu.SemaphoreType.DMA,
          pltpu.SemaphoreType.DMA,
      ],
  )
  def k(mem_h, val_h, idx_h, out_h, idx_v, tag, cbuf, l0, l1, l2, l3, s0, s1,
        s2, s3, gsem):
    wid = lax.axis_index("s") * _NC + lax.axis_index("c")
    lo = pl.multiple_of(wid * _R, 8)
    hi = jnp.where(wid == _NW - 1, _M, lo + _R)
    lsems = (l0, l1, l2, l3)
    stsems = (s0, s1, s2, s3)

    def mk_load(c):
      return pltpu.make_async_copy(
          mem_h.at[pl.ds(lo + c * _CR, _CR)], cbuf.at[c % _NBUF],
          lsems[c % _NBUF])

    def mk_store(c):
      return pltpu.make_async_copy(
          cbuf.at[c % _NBUF], out_h.at[pl.ds(lo + c * _CR, _CR)],
          stsems[c % _NBUF])

    loads = [mk_load(0), mk_load(1)]
    loads[0].start()
    loads[1].start()

    pltpu.sync_copy(idx_h, idx_v)
    iota = lax.iota(jnp.int32, 16)
    minus1 = jnp.full((16,), -1, jnp.int32)

    def init_body(i, carry):
      tag[pl.ds(i * 16, 16)] = minus1
      return carry

    lax.fori_loop(0, _RLAST // 16, init_body, 0)

    def tag_body(i, carry):
      v = idx_v[pl.ds(i * 16, 16)]
      m = (v >= lo) & (v < hi)
      local = jnp.where(m, v - lo, 0)
      plsc.store_scatter(tag, [local], iota + i * 16, mask=m)
      return carry

    lax.fori_loop(0, _NV, tag_body, 0)

    def overlay(c, buf):
      g = pltpu.make_async_copy(
          val_h.at[plsc.Indices(
              tag.at[pl.ds(c * _CR, _CR)], ignored_value=-1)], buf, gsem)
      g.start()
      g.wait()

    stores = [None] * _NCP
    for c in range(_NCP):
      b = c % _NBUF
      loads[c].wait()
      overlay(c, cbuf.at[b])
      st = mk_store(c)
      st.start()
      stores[c] = st
      if c + 2 < _NCP:
        if c >= 2:
          stores[c - 2].wait()
        ld = mk_load(c + 2)
        ld.start()
        loads.append(ld)
    stores[_NCP - 2].wait()
    stores[_NCP - 1].wait()

    @pl.when(wid == _NW - 1)
    def _tail():
      pltpu.sync_copy(mem_h.at[pl.ds(_M - _REM, _REM)],
                      cbuf.at[0, pl.ds(0, _REM)])
      g = pltpu.make_async_copy(
          val_h.at[plsc.Indices(
              tag.at[pl.ds(_R, _REM)], ignored_value=-1)],
          cbuf.at[0, pl.ds(0, _REM)], gsem)
      g.start()
      g.wait()
      pltpu.sync_copy(cbuf.at[0, pl.ds(0, _REM)],
                      out_h.at[pl.ds(_M - _REM, _REM)])

  return k(mem, val, idx)


def kernel(mem, val, idx):
  return _sc_scatter_update(mem, val, idx.astype(jnp.int32))
